# Initial kernel scaffold; baseline (speedup 1.0000x reference)
import jax
import jax.numpy as jnp
from jax.experimental import pallas as pl


def kernel(features, descriptors, idx_i, idx_j, Wg, Wi, bi, Wj, bj, ri_W1, ri_b1, ri_W2, ri_b2, Wd, bd, u, ra_W1, ra_b1, ra_W2, ra_b2):
    raise NotImplementedError("write your pallas kernel here")


# speedup vs baseline: 2.4960x; 2.4960x over previous
import functools

import jax
import jax.numpy as jnp
from jax import lax
from jax.experimental import pallas as pl
from jax.experimental.pallas import tpu as pltpu
from jax.experimental.pallas import tpu_sc as plsc

N = 10000
E = 320000
D = 128
R = 64

NC = 2
NS = 16
NW = NC * NS
EPW = E // NW
CHUNK = 80
NCHUNK = EPW // CHUNK
RPT = 624
REM = N - NS * RPT
ZB = 208

BN = 2000
BE = 8000


def _silu(x):
    return x * jax.nn.sigmoid(x)


def _a1_body(f_ref, wiT_ref, bi_ref, wjT_ref, bj_ref, xi_ref, xjd_ref):
    xa = _silu(f_ref[...])
    wiT = wiT_ref[...]
    wjT = wjT_ref[...]
    xi_ref[...] = _silu(jnp.dot(xa, wiT, preferred_element_type=jnp.float32)
                        + bi_ref[...])
    xjd_ref[...] = _silu(jnp.dot(xa, wjT, preferred_element_type=jnp.float32)
                         + bj_ref[...])


def _stage_a1(features, wiT, bi, wjT, bj):
    grid = (N // BN,)
    return pl.pallas_call(
        _a1_body,
        grid=grid,
        in_specs=[
            pl.BlockSpec((BN, D), lambda i: (i, 0)),
            pl.BlockSpec((D, D), lambda i: (0, 0)),
            pl.BlockSpec((1, D), lambda i: (0, 0)),
            pl.BlockSpec((D, D), lambda i: (0, 0)),
            pl.BlockSpec((1, D), lambda i: (0, 0)),
        ],
        out_specs=[
            pl.BlockSpec((BN, D), lambda i: (i, 0)),
            pl.BlockSpec((BN, D), lambda i: (i, 0)),
        ],
        out_shape=[
            jax.ShapeDtypeStruct((N, D), jnp.float32),
            jax.ShapeDtypeStruct((N, D), jnp.float32),
        ],
    )(features, wiT, bi, wjT, bj)


def _a2_body(d_ref, wgT_ref, g_ref):
    g_ref[...] = jnp.dot(d_ref[...], wgT_ref[...],
                         preferred_element_type=jnp.float32)


def _stage_a2(descriptors, wgT):
    grid = (E // BE,)
    return pl.pallas_call(
        _a2_body,
        grid=grid,
        in_specs=[
            pl.BlockSpec((BE, R), lambda i: (i, 0)),
            pl.BlockSpec((R, D), lambda i: (0, 0)),
        ],
        out_specs=pl.BlockSpec((BE, D), lambda i: (i, 0)),
        out_shape=jax.ShapeDtypeStruct((E, D), jnp.float32),
    )(descriptors, wgT)


def _sc_edge_body(g_hbm, xjd_hbm, idxj_hbm, idxi_hbm, out_hbm,
                  g_v, rows_v, zero_v, idxj_v, idxi_v, acc, sem):
    c = lax.axis_index("c")
    s = lax.axis_index("s")

    def _zrow(e, carry):
        for q in range(D // 16):
            zero_v[e, pl.ds(q * 16, 16)] = jnp.zeros((16,), jnp.float32)
        return carry
    lax.fori_loop(0, ZB, _zrow, 0)
    for k in range(RPT // ZB):
        pltpu.sync_copy(zero_v, acc.at[pl.ds(s * RPT + k * ZB, ZB)])

    @pl.when(s == 0)
    def _zero_rem():
        pltpu.sync_copy(zero_v.at[pl.ds(0, REM)], acc.at[pl.ds(NS * RPT, REM)])

    plsc.subcore_barrier()

    wbase = (c * NS + s) * EPW

    def _chunk(k, carry):
        base = wbase + k * CHUNK
        pltpu.sync_copy(idxj_hbm.at[pl.ds(base, CHUNK)], idxj_v)
        pltpu.sync_copy(idxi_hbm.at[pl.ds(base, CHUNK)], idxi_v)
        pltpu.sync_copy(g_hbm.at[pl.ds(base, CHUNK)], g_v)
        pltpu.async_copy(xjd_hbm.at[idxj_v], rows_v, sem).wait()

        def _mul(e, cc):
            for q in range(D // 16):
                sl = pl.ds(q * 16, 16)
                g_v[e, sl] = g_v[e, sl] * rows_v[e, sl]
            return cc
        lax.fori_loop(0, CHUNK, _mul, 0)
        pltpu.sync_copy(g_v, acc.at[idxi_v], add=True)
        return carry

    lax.fori_loop(0, NCHUNK, _chunk, 0)
    plsc.subcore_barrier()

    rbase = s * RPT
    pltpu.sync_copy(acc.at[pl.ds(rbase, RPT)],
                    out_hbm.at[c, pl.ds(rbase, RPT)])

    @pl.when(s == 0)
    def _write_rem():
        pltpu.sync_copy(acc.at[pl.ds(NS * RPT, REM)],
                        out_hbm.at[c, pl.ds(NS * RPT, REM)])


def _stage_b(g, xjd, idx_j, idx_i):
    mesh = plsc.VectorSubcoreMesh(core_axis_name="c", subcore_axis_name="s",
                                  num_cores=NC, num_subcores=NS)
    fn = pl.kernel(
        _sc_edge_body,
        out_type=jax.ShapeDtypeStruct((NC, N, D), jnp.float32),
        mesh=mesh,
        scratch_types=[
            pltpu.VMEM((CHUNK, D), jnp.float32),
            pltpu.VMEM((CHUNK, D), jnp.float32),
            pltpu.VMEM((ZB, D), jnp.float32),
            pltpu.VMEM((CHUNK,), jnp.int32),
            pltpu.VMEM((CHUNK,), jnp.int32),
            pltpu.VMEM_SHARED((N, D), jnp.float32),
            pltpu.SemaphoreType.DMA,
        ],
    )
    return fn(g, xjd, idx_j, idx_i)


def _c_body(xi_ref, p_ref, f_ref,
            riW1T_ref, rib1_ref, riW2T_ref, rib2_ref,
            wdT_ref, bd_ref, u_ref,
            raW1T_ref, rab1_ref, raW2T_ref, rab2_ref,
            out_ref):
    m = xi_ref[...] + p_ref[0] + p_ref[1]
    for i in range(riW1T_ref.shape[0]):
        y = _silu(m)
        t = _silu(jnp.dot(y, riW1T_ref[i], preferred_element_type=jnp.float32)
                  + rib1_ref[i])
        m = m + jnp.dot(t, riW2T_ref[i], preferred_element_type=jnp.float32) \
              + rib2_ref[i]
    m = _silu(m)
    x = u_ref[...] * f_ref[...] \
        + jnp.dot(m, wdT_ref[...], preferred_element_type=jnp.float32) \
        + bd_ref[...]
    for i in range(raW1T_ref.shape[0]):
        y = _silu(x)
        t = _silu(jnp.dot(y, raW1T_ref[i], preferred_element_type=jnp.float32)
                  + rab1_ref[i])
        x = x + jnp.dot(t, raW2T_ref[i], preferred_element_type=jnp.float32) \
              + rab2_ref[i]
    out_ref[...] = x


def _stage_c(xi, p, features, riW1T, rib1, riW2T, rib2, wdT, bd, u,
             raW1T, rab1, raW2T, rab2):
    grid = (N // BN,)
    nri = riW1T.shape[0]
    nra = raW1T.shape[0]
    return pl.pallas_call(
        _c_body,
        grid=grid,
        in_specs=[
            pl.BlockSpec((BN, D), lambda i: (i, 0)),
            pl.BlockSpec((NC, BN, D), lambda i: (0, i, 0)),
            pl.BlockSpec((BN, D), lambda i: (i, 0)),
            pl.BlockSpec((nri, D, D), lambda i: (0, 0, 0)),
            pl.BlockSpec((nri, 1, D), lambda i: (0, 0, 0)),
            pl.BlockSpec((nri, D, D), lambda i: (0, 0, 0)),
            pl.BlockSpec((nri, 1, D), lambda i: (0, 0, 0)),
            pl.BlockSpec((D, D), lambda i: (0, 0)),
            pl.BlockSpec((1, D), lambda i: (0, 0)),
            pl.BlockSpec((1, D), lambda i: (0, 0)),
            pl.BlockSpec((nra, D, D), lambda i: (0, 0, 0)),
            pl.BlockSpec((nra, 1, D), lambda i: (0, 0, 0)),
            pl.BlockSpec((nra, D, D), lambda i: (0, 0, 0)),
            pl.BlockSpec((nra, 1, D), lambda i: (0, 0, 0)),
        ],
        out_specs=pl.BlockSpec((BN, D), lambda i: (i, 0)),
        out_shape=jax.ShapeDtypeStruct((N, D), jnp.float32),
    )(xi, p, features, riW1T, rib1, riW2T, rib2, wdT, bd, u,
      raW1T, rab1, raW2T, rab2)


def kernel(features, descriptors, idx_i, idx_j, Wg, Wi, bi, Wj, bj,
           ri_W1, ri_b1, ri_W2, ri_b2, Wd, bd, u, ra_W1, ra_b1, ra_W2, ra_b2):
    wiT = Wi.T
    wjT = Wj.T
    wgT = Wg.T
    wdT = Wd.T
    riW1T = jnp.swapaxes(ri_W1, 1, 2)
    riW2T = jnp.swapaxes(ri_W2, 1, 2)
    raW1T = jnp.swapaxes(ra_W1, 1, 2)
    raW2T = jnp.swapaxes(ra_W2, 1, 2)
    bi2 = bi.reshape(1, D)
    bj2 = bj.reshape(1, D)
    bd2 = bd.reshape(1, D)
    u2 = u.reshape(1, D)
    rib1 = ri_b1.reshape(-1, 1, D)
    rib2 = ri_b2.reshape(-1, 1, D)
    rab1 = ra_b1.reshape(-1, 1, D)
    rab2 = ra_b2.reshape(-1, 1, D)

    xi, xjd = _stage_a1(features, wiT, bi2, wjT, bj2)
    g = _stage_a2(descriptors, wgT)
    p = _stage_b(g, xjd, idx_j, idx_i)
    return _stage_c(xi, p, features, riW1T, rib1, riW2T, rib2, wdT, bd2, u2,
                    raW1T, rab1, raW2T, rab2)


# speedup vs baseline: 4.2172x; 1.6896x over previous
import functools

import jax
import jax.numpy as jnp
from jax import lax
from jax.experimental import pallas as pl
from jax.experimental.pallas import tpu as pltpu
from jax.experimental.pallas import tpu_sc as plsc

N = 10000
E = 320000
D = 128
R = 64

NC = 2
NS = 16
NW = NC * NS
EPW = E // NW
CHUNK = 64
NCHUNK = 156
TAIL = EPW - NCHUNK * CHUNK
RPT = 624
REM = N - NS * RPT
ZB = 48

BN = 2000
BE = 8000


def _silu(x):
    return x * jax.nn.sigmoid(x)


def _a1_body(f_ref, wiT_ref, bi_ref, wjT_ref, bj_ref, xi_ref, xjd_ref):
    xa = _silu(f_ref[...])
    wiT = wiT_ref[...]
    wjT = wjT_ref[...]
    xi_ref[...] = _silu(jnp.dot(xa, wiT, preferred_element_type=jnp.float32)
                        + bi_ref[...])
    xjd_ref[...] = _silu(jnp.dot(xa, wjT, preferred_element_type=jnp.float32)
                         + bj_ref[...])


def _stage_a1(features, wiT, bi, wjT, bj):
    grid = (N // BN,)
    return pl.pallas_call(
        _a1_body,
        grid=grid,
        in_specs=[
            pl.BlockSpec((BN, D), lambda i: (i, 0)),
            pl.BlockSpec((D, D), lambda i: (0, 0)),
            pl.BlockSpec((1, D), lambda i: (0, 0)),
            pl.BlockSpec((D, D), lambda i: (0, 0)),
            pl.BlockSpec((1, D), lambda i: (0, 0)),
        ],
        out_specs=[
            pl.BlockSpec((BN, D), lambda i: (i, 0)),
            pl.BlockSpec((BN, D), lambda i: (i, 0)),
        ],
        out_shape=[
            jax.ShapeDtypeStruct((N, D), jnp.float32),
            jax.ShapeDtypeStruct((N, D), jnp.float32),
        ],
    )(features, wiT, bi, wjT, bj)


def _a2_body(d_ref, wgT_ref, g_ref):
    g_ref[...] = jnp.dot(d_ref[...], wgT_ref[...],
                         preferred_element_type=jnp.float32)


def _stage_a2(descriptors, wgT):
    grid = (E // BE,)
    return pl.pallas_call(
        _a2_body,
        grid=grid,
        in_specs=[
            pl.BlockSpec((BE, R), lambda i: (i, 0)),
            pl.BlockSpec((R, D), lambda i: (0, 0)),
        ],
        out_specs=pl.BlockSpec((BE, D), lambda i: (i, 0)),
        out_shape=jax.ShapeDtypeStruct((E, D), jnp.float32),
    )(descriptors, wgT)


def _sc_edge_body(g_hbm, xjd_hbm, idxj_hbm, idxi_hbm, out_hbm,
                  g_v0, g_v1, rows_v0, rows_v1, zero_v,
                  idxj_v0, idxj_v1, idxi_v0, idxi_v1,
                  jt, it, gt, rt, acc,
                  sg0, sg1, sr0, sr1, sj0, sj1, si0, si1):
    c = lax.axis_index("c")
    s = lax.axis_index("s")

    gv = (g_v0, g_v1)
    rv = (rows_v0, rows_v1)
    jv = (idxj_v0, idxj_v1)
    iv = (idxi_v0, idxi_v1)
    sg = (sg0, sg1)
    sr = (sr0, sr1)
    sj = (sj0, sj1)
    si = (si0, si1)

    def _zrow(e, carry):
        for q in range(D // 16):
            zero_v[e, pl.ds(q * 16, 16)] = jnp.zeros((16,), jnp.float32)
        return carry
    lax.fori_loop(0, ZB, _zrow, 0)
    for k in range(RPT // ZB):
        pltpu.sync_copy(zero_v, acc.at[pl.ds(s * RPT + k * ZB, ZB)])

    @pl.when(s == 0)
    def _zero_rem():
        pltpu.sync_copy(zero_v.at[pl.ds(0, REM)], acc.at[pl.ds(NS * RPT, REM)])

    plsc.subcore_barrier()

    wbase = (c * NS + s) * EPW

    def _ds(kk):
        kkw = jnp.minimum(kk, NCHUNK - 1)
        return pl.ds(wbase + kkw * CHUNK, CHUNK)

    def _mul_chunk(b):
        def _mul(e, cc):
            for q in range(D // 16):
                sl = pl.ds(q * 16, 16)
                gv[b][e, sl] = gv[b][e, sl] * rv[b][e, sl]
            return cc
        lax.fori_loop(0, CHUNK, _mul, 0)

    def _step(k, b):
        nb = 1 - b
        pltpu.make_async_copy(idxj_hbm.at[_ds(k + 1)], jv[nb], sj[nb]).wait()
        pltpu.async_copy(g_hbm.at[_ds(k + 1)], gv[nb], sg[nb])
        pltpu.async_copy(xjd_hbm.at[jv[nb]], rv[nb], sr[nb])
        pltpu.make_async_copy(g_hbm.at[_ds(k)], gv[b], sg[b]).wait()
        pltpu.make_async_copy(xjd_hbm.at[jv[b]], rv[b], sr[b]).wait()
        pltpu.async_copy(idxj_hbm.at[_ds(k + 2)], jv[b], sj[b])
        _mul_chunk(b)
        pltpu.make_async_copy(idxi_hbm.at[_ds(k)], iv[b], si[b]).wait()
        pltpu.sync_copy(gv[b], acc.at[iv[b]], add=True)
        pltpu.async_copy(idxi_hbm.at[_ds(k + 2)], iv[b], si[b])

    pltpu.async_copy(idxj_hbm.at[_ds(0)], jv[0], sj[0])
    pltpu.async_copy(idxi_hbm.at[_ds(0)], iv[0], si[0])
    pltpu.async_copy(idxj_hbm.at[_ds(1)], jv[1], sj[1])
    pltpu.async_copy(idxi_hbm.at[_ds(1)], iv[1], si[1])
    pltpu.make_async_copy(idxj_hbm.at[_ds(0)], jv[0], sj[0]).wait()
    pltpu.async_copy(g_hbm.at[_ds(0)], gv[0], sg[0])
    pltpu.async_copy(xjd_hbm.at[jv[0]], rv[0], sr[0])

    def _pair(t, carry):
        _step(2 * t, 0)
        _step(2 * t + 1, 1)
        return carry
    lax.fori_loop(0, NCHUNK // 2, _pair, 0)

    pltpu.make_async_copy(g_hbm.at[_ds(0)], gv[0], sg[0]).wait()
    pltpu.make_async_copy(xjd_hbm.at[jv[0]], rv[0], sr[0]).wait()
    pltpu.make_async_copy(idxj_hbm.at[_ds(0)], jv[1], sj[1]).wait()
    pltpu.make_async_copy(idxi_hbm.at[_ds(0)], iv[0], si[0]).wait()
    pltpu.make_async_copy(idxi_hbm.at[_ds(0)], iv[1], si[1]).wait()

    wtail = wbase + NCHUNK * CHUNK
    pltpu.sync_copy(idxj_hbm.at[pl.ds(wtail, TAIL)], jt)
    pltpu.sync_copy(idxi_hbm.at[pl.ds(wtail, TAIL)], it)
    pltpu.sync_copy(g_hbm.at[pl.ds(wtail, TAIL)], gt)
    pltpu.async_copy(xjd_hbm.at[jt], rt, sr0).wait()

    def _mul_tail(e, cc):
        for q in range(D // 16):
            sl = pl.ds(q * 16, 16)
            gt[e, sl] = gt[e, sl] * rt[e, sl]
        return cc
    lax.fori_loop(0, TAIL, _mul_tail, 0)
    pltpu.sync_copy(gt, acc.at[it], add=True)

    plsc.subcore_barrier()

    rbase = s * RPT
    pltpu.sync_copy(acc.at[pl.ds(rbase, RPT)],
                    out_hbm.at[c, pl.ds(rbase, RPT)])

    @pl.when(s == 0)
    def _write_rem():
        pltpu.sync_copy(acc.at[pl.ds(NS * RPT, REM)],
                        out_hbm.at[c, pl.ds(NS * RPT, REM)])


def _stage_b(g, xjd, idx_j, idx_i):
    mesh = plsc.VectorSubcoreMesh(core_axis_name="c", subcore_axis_name="s",
                                  num_cores=NC, num_subcores=NS)
    fn = pl.kernel(
        _sc_edge_body,
        out_type=jax.ShapeDtypeStruct((NC, N, D), jnp.float32),
        mesh=mesh,
        scratch_types=[
            pltpu.VMEM((CHUNK, D), jnp.float32),
            pltpu.VMEM((CHUNK, D), jnp.float32),
            pltpu.VMEM((CHUNK, D), jnp.float32),
            pltpu.VMEM((CHUNK, D), jnp.float32),
            pltpu.VMEM((ZB, D), jnp.float32),
            pltpu.VMEM((CHUNK,), jnp.int32),
            pltpu.VMEM((CHUNK,), jnp.int32),
            pltpu.VMEM((CHUNK,), jnp.int32),
            pltpu.VMEM((CHUNK,), jnp.int32),
            pltpu.VMEM((TAIL,), jnp.int32),
            pltpu.VMEM((TAIL,), jnp.int32),
            pltpu.VMEM((TAIL, D), jnp.float32),
            pltpu.VMEM((TAIL, D), jnp.float32),
            pltpu.VMEM_SHARED((N, D), jnp.float32),
            pltpu.SemaphoreType.DMA,
            pltpu.SemaphoreType.DMA,
            pltpu.SemaphoreType.DMA,
            pltpu.SemaphoreType.DMA,
            pltpu.SemaphoreType.DMA,
            pltpu.SemaphoreType.DMA,
            pltpu.SemaphoreType.DMA,
            pltpu.SemaphoreType.DMA,
        ],
    )
    return fn(g, xjd, idx_j, idx_i)


def _c_body(xi_ref, p_ref, f_ref,
            riW1T_ref, rib1_ref, riW2T_ref, rib2_ref,
            wdT_ref, bd_ref, u_ref,
            raW1T_ref, rab1_ref, raW2T_ref, rab2_ref,
            out_ref):
    m = xi_ref[...] + p_ref[0] + p_ref[1]
    for i in range(riW1T_ref.shape[0]):
        y = _silu(m)
        t = _silu(jnp.dot(y, riW1T_ref[i], preferred_element_type=jnp.float32)
                  + rib1_ref[i])
        m = m + jnp.dot(t, riW2T_ref[i], preferred_element_type=jnp.float32) \
              + rib2_ref[i]
    m = _silu(m)
    x = u_ref[...] * f_ref[...] \
        + jnp.dot(m, wdT_ref[...], preferred_element_type=jnp.float32) \
        + bd_ref[...]
    for i in range(raW1T_ref.shape[0]):
        y = _silu(x)
        t = _silu(jnp.dot(y, raW1T_ref[i], preferred_element_type=jnp.float32)
                  + rab1_ref[i])
        x = x + jnp.dot(t, raW2T_ref[i], preferred_element_type=jnp.float32) \
              + rab2_ref[i]
    out_ref[...] = x


def _stage_c(xi, p, features, riW1T, rib1, riW2T, rib2, wdT, bd, u,
             raW1T, rab1, raW2T, rab2):
    grid = (N // BN,)
    nri = riW1T.shape[0]
    nra = raW1T.shape[0]
    return pl.pallas_call(
        _c_body,
        grid=grid,
        in_specs=[
            pl.BlockSpec((BN, D), lambda i: (i, 0)),
            pl.BlockSpec((NC, BN, D), lambda i: (0, i, 0)),
            pl.BlockSpec((BN, D), lambda i: (i, 0)),
            pl.BlockSpec((nri, D, D), lambda i: (0, 0, 0)),
            pl.BlockSpec((nri, 1, D), lambda i: (0, 0, 0)),
            pl.BlockSpec((nri, D, D), lambda i: (0, 0, 0)),
            pl.BlockSpec((nri, 1, D), lambda i: (0, 0, 0)),
            pl.BlockSpec((D, D), lambda i: (0, 0)),
            pl.BlockSpec((1, D), lambda i: (0, 0)),
            pl.BlockSpec((1, D), lambda i: (0, 0)),
            pl.BlockSpec((nra, D, D), lambda i: (0, 0, 0)),
            pl.BlockSpec((nra, 1, D), lambda i: (0, 0, 0)),
            pl.BlockSpec((nra, D, D), lambda i: (0, 0, 0)),
            pl.BlockSpec((nra, 1, D), lambda i: (0, 0, 0)),
        ],
        out_specs=pl.BlockSpec((BN, D), lambda i: (i, 0)),
        out_shape=jax.ShapeDtypeStruct((N, D), jnp.float32),
    )(xi, p, features, riW1T, rib1, riW2T, rib2, wdT, bd, u,
      raW1T, rab1, raW2T, rab2)


def kernel(features, descriptors, idx_i, idx_j, Wg, Wi, bi, Wj, bj,
           ri_W1, ri_b1, ri_W2, ri_b2, Wd, bd, u, ra_W1, ra_b1, ra_W2, ra_b2):
    wiT = Wi.T
    wjT = Wj.T
    wgT = Wg.T
    wdT = Wd.T
    riW1T = jnp.swapaxes(ri_W1, 1, 2)
    riW2T = jnp.swapaxes(ri_W2, 1, 2)
    raW1T = jnp.swapaxes(ra_W1, 1, 2)
    raW2T = jnp.swapaxes(ra_W2, 1, 2)
    bi2 = bi.reshape(1, D)
    bj2 = bj.reshape(1, D)
    bd2 = bd.reshape(1, D)
    u2 = u.reshape(1, D)
    rib1 = ri_b1.reshape(-1, 1, D)
    rib2 = ri_b2.reshape(-1, 1, D)
    rab1 = ra_b1.reshape(-1, 1, D)
    rab2 = ra_b2.reshape(-1, 1, D)

    xi, xjd = _stage_a1(features, wiT, bi2, wjT, bj2)
    g = _stage_a2(descriptors, wgT)
    p = _stage_b(g, xjd, idx_j, idx_i)
    return _stage_c(xi, p, features, riW1T, rib1, riW2T, rib2, wdT, bd2, u2,
                    raW1T, rab1, raW2T, rab2)


# speedup vs baseline: 4.3294x; 1.0266x over previous
import functools

import numpy as np

import jax
import jax.numpy as jnp
from jax import lax
from jax.experimental import pallas as pl
from jax.experimental.pallas import tpu as pltpu
from jax.experimental.pallas import tpu_sc as plsc

N = 10000
E = 320000
D = 128
R = 64

NC = 2
NS = 16
NW = NC * NS
EPW = E // NW
CHUNK = 72
NCHUNK = 138
TAIL = EPW - NCHUNK * CHUNK
RPT = 624
REM = N - NS * RPT

BN = 2000
BE = 8000


def _silu(x):
    return x * jax.nn.sigmoid(x)


def _pack_bf16_pair(lo_f32, hi_f32):
    hi_bits = jax.lax.bitcast_convert_type(
        hi_f32.astype(jnp.bfloat16).astype(jnp.float32), jnp.int32)
    lo_bits = jax.lax.shift_right_logical(
        jax.lax.bitcast_convert_type(
            lo_f32.astype(jnp.bfloat16).astype(jnp.float32), jnp.int32),
        jnp.int32(16))
    return hi_bits | lo_bits


def _a1_body(f_ref, wiT_ref, bi_ref, wjT_ref, bj_ref, xi_ref, xjd_ref):
    xa = _silu(f_ref[...])
    xi_ref[...] = _silu(jnp.dot(xa, wiT_ref[...],
                                preferred_element_type=jnp.float32)
                        + bi_ref[...])
    xjd_ref[...] = _silu(jnp.dot(xa, wjT_ref[...],
                                 preferred_element_type=jnp.float32)
                         + bj_ref[...])


def _stage_a1(features, wiT, bi, wjT, bj):
    grid = (N // BN,)
    return pl.pallas_call(
        _a1_body,
        grid=grid,
        in_specs=[
            pl.BlockSpec((BN, D), lambda i: (i, 0)),
            pl.BlockSpec((D, D), lambda i: (0, 0)),
            pl.BlockSpec((1, D), lambda i: (0, 0)),
            pl.BlockSpec((D, D), lambda i: (0, 0)),
            pl.BlockSpec((1, D), lambda i: (0, 0)),
        ],
        out_specs=[
            pl.BlockSpec((BN, D), lambda i: (i, 0)),
            pl.BlockSpec((BN, D), lambda i: (i, 0)),
        ],
        out_shape=[
            jax.ShapeDtypeStruct((N, D), jnp.float32),
            jax.ShapeDtypeStruct((N, D), jnp.float32),
        ],
    )(features, wiT, bi, wjT, bj)


def _a2_body(d_ref, wgTlo_ref, wgThi_ref, g_ref):
    d = d_ref[...]
    glo = jnp.dot(d, wgTlo_ref[...], preferred_element_type=jnp.float32)
    ghi = jnp.dot(d, wgThi_ref[...], preferred_element_type=jnp.float32)
    g_ref[...] = _pack_bf16_pair(glo, ghi)


def _stage_a2(descriptors, wgTlo, wgThi):
    grid = (E // BE,)
    h = D // 2
    return pl.pallas_call(
        _a2_body,
        grid=grid,
        in_specs=[
            pl.BlockSpec((BE, R), lambda i: (i, 0)),
            pl.BlockSpec((R, h), lambda i: (0, 0)),
            pl.BlockSpec((R, h), lambda i: (0, 0)),
        ],
        out_specs=pl.BlockSpec((BE, h), lambda i: (i, 0)),
        out_shape=jax.ShapeDtypeStruct((E, h), jnp.int32),
    )(descriptors, wgTlo, wgThi)


def _sc_edge_body(g_hbm, xjd_hbm, idxj_hbm, idxi_hbm, out_hbm,
                  g_v0, g_v1, rows_v0, rows_v1, prod,
                  idxj_v0, idxj_v1, idxi_v0, idxi_v1,
                  jt, it, acc,
                  sg0, sg1, sr0, sr1, sj0, sj1, si0, si1):
    c = lax.axis_index("c")
    s = lax.axis_index("s")

    gv = (g_v0, g_v1)
    rv = (rows_v0, rows_v1)
    jv = (idxj_v0, idxj_v1)
    iv = (idxi_v0, idxi_v1)
    sg = (sg0, sg1)
    sr = (sr0, sr1)
    sj = (sj0, sj1)
    si = (si0, si1)

    def _zrow(e, carry):
        for q in range(D // 16):
            prod[e, pl.ds(q * 16, 16)] = jnp.zeros((16,), jnp.float32)
        return carry
    lax.fori_loop(0, CHUNK, _zrow, 0)
    for k in range(RPT // CHUNK):
        pltpu.sync_copy(prod, acc.at[pl.ds(s * RPT + k * CHUNK, CHUNK)])
    nfull = (RPT // CHUNK) * CHUNK
    pltpu.sync_copy(prod.at[pl.ds(0, RPT - nfull)],
                    acc.at[pl.ds(s * RPT + nfull, RPT - nfull)])

    @pl.when(s == 0)
    def _zero_rem():
        pltpu.sync_copy(prod.at[pl.ds(0, REM)], acc.at[pl.ds(NS * RPT, REM)])

    plsc.subcore_barrier()

    wbase = (c * NS + s) * EPW

    def _ds(kk):
        kkw = jnp.minimum(kk, NCHUNK - 1)
        return pl.ds(wbase + kkw * CHUNK, CHUNK)

    himask = jax.lax.broadcast(jnp.int32(-65536), (16,))

    def _mul_edge(gref, rref, dst_e, e):
        for q in range(D // 32):
            ds16 = pl.ds(q * 16, 16)
            gw = gref[e, ds16]
            glo = jax.lax.bitcast_convert_type(
                jax.lax.shift_left(gw, 16), jnp.float32)
            ghi = jax.lax.bitcast_convert_type(gw & himask, jnp.float32)
            rlo = rref[e, pl.ds(q * 16, 16)]
            rhi = rref[e, pl.ds(64 + q * 16, 16)]
            prod[dst_e, pl.ds(q * 16, 16)] = glo * rlo
            prod[dst_e, pl.ds(64 + q * 16, 16)] = ghi * rhi

    def _mul_chunk(b):
        def _mul(e, cc):
            _mul_edge(gv[b], rv[b], e, e)
            return cc
        lax.fori_loop(0, CHUNK, _mul, 0)

    def _step(k, b):
        nb = 1 - b
        pltpu.make_async_copy(idxj_hbm.at[_ds(k + 1)], jv[nb], sj[nb]).wait()
        pltpu.async_copy(g_hbm.at[_ds(k + 1)], gv[nb], sg[nb])
        pltpu.async_copy(xjd_hbm.at[jv[nb]], rv[nb], sr[nb])
        pltpu.make_async_copy(g_hbm.at[_ds(k)], gv[b], sg[b]).wait()
        pltpu.make_async_copy(xjd_hbm.at[jv[b]], rv[b], sr[b]).wait()
        pltpu.async_copy(idxj_hbm.at[_ds(k + 2)], jv[b], sj[b])
        _mul_chunk(b)
        pltpu.make_async_copy(idxi_hbm.at[_ds(k)], iv[b], si[b]).wait()
        pltpu.sync_copy(prod, acc.at[iv[b]], add=True)
        pltpu.async_copy(idxi_hbm.at[_ds(k + 2)], iv[b], si[b])

    pltpu.async_copy(idxj_hbm.at[_ds(0)], jv[0], sj[0])
    pltpu.async_copy(idxi_hbm.at[_ds(0)], iv[0], si[0])
    pltpu.async_copy(idxj_hbm.at[_ds(1)], jv[1], sj[1])
    pltpu.async_copy(idxi_hbm.at[_ds(1)], iv[1], si[1])
    pltpu.make_async_copy(idxj_hbm.at[_ds(0)], jv[0], sj[0]).wait()
    pltpu.async_copy(g_hbm.at[_ds(0)], gv[0], sg[0])
    pltpu.async_copy(xjd_hbm.at[jv[0]], rv[0], sr[0])

    def _pair(t, carry):
        _step(2 * t, 0)
        _step(2 * t + 1, 1)
        return carry
    lax.fori_loop(0, NCHUNK // 2, _pair, 0)

    pltpu.make_async_copy(g_hbm.at[_ds(0)], gv[0], sg[0]).wait()
    pltpu.make_async_copy(xjd_hbm.at[jv[0]], rv[0], sr[0]).wait()
    pltpu.make_async_copy(idxj_hbm.at[_ds(0)], jv[1], sj[1]).wait()
    pltpu.make_async_copy(idxi_hbm.at[_ds(0)], iv[0], si[0]).wait()
    pltpu.make_async_copy(idxi_hbm.at[_ds(0)], iv[1], si[1]).wait()

    wtail = wbase + NCHUNK * CHUNK
    pltpu.sync_copy(idxj_hbm.at[pl.ds(wtail, TAIL)], jt)
    pltpu.sync_copy(idxi_hbm.at[pl.ds(wtail, TAIL)], it)
    pltpu.sync_copy(g_hbm.at[pl.ds(wtail, TAIL)], gv[0].at[pl.ds(0, TAIL)])
    pltpu.async_copy(xjd_hbm.at[jt], rv[0].at[pl.ds(0, TAIL)], sr0).wait()

    def _mul_tail(e, cc):
        _mul_edge(gv[0], rv[0], e, e)
        return cc
    lax.fori_loop(0, TAIL, _mul_tail, 0)
    pltpu.sync_copy(prod.at[pl.ds(0, TAIL)], acc.at[it], add=True)

    plsc.subcore_barrier()

    rbase = s * RPT
    pltpu.sync_copy(acc.at[pl.ds(rbase, RPT)],
                    out_hbm.at[c, pl.ds(rbase, RPT)])

    @pl.when(s == 0)
    def _write_rem():
        pltpu.sync_copy(acc.at[pl.ds(NS * RPT, REM)],
                        out_hbm.at[c, pl.ds(NS * RPT, REM)])


def _stage_b(g, xjd, idx_j, idx_i):
    mesh = plsc.VectorSubcoreMesh(core_axis_name="c", subcore_axis_name="s",
                                  num_cores=NC, num_subcores=NS)
    fn = pl.kernel(
        _sc_edge_body,
        out_type=jax.ShapeDtypeStruct((NC, N, D), jnp.float32),
        mesh=mesh,
        scratch_types=[
            pltpu.VMEM((CHUNK, D // 2), jnp.int32),
            pltpu.VMEM((CHUNK, D // 2), jnp.int32),
            pltpu.VMEM((CHUNK, D), jnp.float32),
            pltpu.VMEM((CHUNK, D), jnp.float32),
            pltpu.VMEM((CHUNK, D), jnp.float32),
            pltpu.VMEM((CHUNK,), jnp.int32),
            pltpu.VMEM((CHUNK,), jnp.int32),
            pltpu.VMEM((CHUNK,), jnp.int32),
            pltpu.VMEM((CHUNK,), jnp.int32),
            pltpu.VMEM((TAIL,), jnp.int32),
            pltpu.VMEM((TAIL,), jnp.int32),
            pltpu.VMEM_SHARED((N, D), jnp.float32),
            pltpu.SemaphoreType.DMA,
            pltpu.SemaphoreType.DMA,
            pltpu.SemaphoreType.DMA,
            pltpu.SemaphoreType.DMA,
            pltpu.SemaphoreType.DMA,
            pltpu.SemaphoreType.DMA,
            pltpu.SemaphoreType.DMA,
            pltpu.SemaphoreType.DMA,
        ],
    )
    return fn(g, xjd, idx_j, idx_i)


def _c_body(xi_ref, p_ref, f_ref,
            riW1T_ref, rib1_ref, riW2T_ref, rib2_ref,
            wdT_ref, bd_ref, u_ref,
            raW1T_ref, rab1_ref, raW2T_ref, rab2_ref,
            out_ref):
    m = xi_ref[...] + p_ref[0] + p_ref[1]
    for i in range(riW1T_ref.shape[0]):
        y = _silu(m)
        t = _silu(jnp.dot(y, riW1T_ref[i], preferred_element_type=jnp.float32)
                  + rib1_ref[i])
        m = m + jnp.dot(t, riW2T_ref[i], preferred_element_type=jnp.float32) \
              + rib2_ref[i]
    m = _silu(m)
    x = u_ref[...] * f_ref[...] \
        + jnp.dot(m, wdT_ref[...], preferred_element_type=jnp.float32) \
        + bd_ref[...]
    for i in range(raW1T_ref.shape[0]):
        y = _silu(x)
        t = _silu(jnp.dot(y, raW1T_ref[i], preferred_element_type=jnp.float32)
                  + rab1_ref[i])
        x = x + jnp.dot(t, raW2T_ref[i], preferred_element_type=jnp.float32) \
              + rab2_ref[i]
    out_ref[...] = x


def _stage_c(xi, p, features, riW1T, rib1, riW2T, rib2, wdT, bd, u,
             raW1T, rab1, raW2T, rab2):
    grid = (N // BN,)
    nri = riW1T.shape[0]
    nra = raW1T.shape[0]
    return pl.pallas_call(
        _c_body,
        grid=grid,
        in_specs=[
            pl.BlockSpec((BN, D), lambda i: (i, 0)),
            pl.BlockSpec((NC, BN, D), lambda i: (0, i, 0)),
            pl.BlockSpec((BN, D), lambda i: (i, 0)),
            pl.BlockSpec((nri, D, D), lambda i: (0, 0, 0)),
            pl.BlockSpec((nri, 1, D), lambda i: (0, 0, 0)),
            pl.BlockSpec((nri, D, D), lambda i: (0, 0, 0)),
            pl.BlockSpec((nri, 1, D), lambda i: (0, 0, 0)),
            pl.BlockSpec((D, D), lambda i: (0, 0)),
            pl.BlockSpec((1, D), lambda i: (0, 0)),
            pl.BlockSpec((1, D), lambda i: (0, 0)),
            pl.BlockSpec((nra, D, D), lambda i: (0, 0, 0)),
            pl.BlockSpec((nra, 1, D), lambda i: (0, 0, 0)),
            pl.BlockSpec((nra, D, D), lambda i: (0, 0, 0)),
            pl.BlockSpec((nra, 1, D), lambda i: (0, 0, 0)),
        ],
        out_specs=pl.BlockSpec((BN, D), lambda i: (i, 0)),
        out_shape=jax.ShapeDtypeStruct((N, D), jnp.float32),
    )(xi, p, features, riW1T, rib1, riW2T, rib2, wdT, bd, u,
      raW1T, rab1, raW2T, rab2)


def kernel(features, descriptors, idx_i, idx_j, Wg, Wi, bi, Wj, bj,
           ri_W1, ri_b1, ri_W2, ri_b2, Wd, bd, u, ra_W1, ra_b1, ra_W2, ra_b2):
    h = D // 2
    wiT = Wi.T
    wjT = Wj.T
    wgT = Wg.T
    wdT = Wd.T
    riW1T = jnp.swapaxes(ri_W1, 1, 2)
    riW2T = jnp.swapaxes(ri_W2, 1, 2)
    raW1T = jnp.swapaxes(ra_W1, 1, 2)
    raW2T = jnp.swapaxes(ra_W2, 1, 2)
    bi2 = bi.reshape(1, D)
    bd2 = bd.reshape(1, D)
    u2 = u.reshape(1, D)
    rib1 = ri_b1.reshape(-1, 1, D)
    rib2 = ri_b2.reshape(-1, 1, D)
    rab1 = ra_b1.reshape(-1, 1, D)
    rab2 = ra_b2.reshape(-1, 1, D)

    xi, xjd = _stage_a1(features, wiT, bi2, wjT, bj.reshape(1, D))
    g = _stage_a2(descriptors, wgT[:, :h], wgT[:, h:])
    p = _stage_b(g, xjd, idx_j, idx_i)
    return _stage_c(xi, p, features, riW1T, rib1, riW2T, rib2, wdT, bd2, u2,
                    raW1T, rab1, raW2T, rab2)


# speedup vs baseline: 6.1419x; 1.4186x over previous
import functools

import numpy as np

import jax
import jax.numpy as jnp
from jax import lax
from jax.experimental import pallas as pl
from jax.experimental.pallas import tpu as pltpu
from jax.experimental.pallas import tpu_sc as plsc

N = 10000
E = 320000
D = 128
R = 64

NC = 2
NS = 16
NW = NC * NS
EPW = E // NW
CHUNK = 72
NCHUNK = 138
TAIL = EPW - NCHUNK * CHUNK
RPT = 624
REM = N - NS * RPT

BN = 2000
BE = 12800


def _silu(x):
    return x * jax.nn.sigmoid(x)


def _dot_t(x, w):
    return jax.lax.dot_general(x, w, (((1,), (1,)), ((), ())),
                               preferred_element_type=jnp.float32)


def _pack_bf16_pair(lo_f32, hi_f32):
    hi_bits = jax.lax.bitcast_convert_type(
        hi_f32.astype(jnp.bfloat16).astype(jnp.float32), jnp.int32)
    lo_bits = jax.lax.shift_right_logical(
        jax.lax.bitcast_convert_type(
            lo_f32.astype(jnp.bfloat16).astype(jnp.float32), jnp.int32),
        jnp.int32(16))
    return hi_bits | lo_bits


def _a1_body(f_ref, wi_ref, bi_ref, wj_ref, bj_ref, xi_ref, xjd_ref):
    xa = _silu(f_ref[...])
    xi_ref[...] = _silu(_dot_t(xa, wi_ref[...]) + bi_ref[...])
    xjd_ref[...] = _silu(_dot_t(xa, wj_ref[...]) + bj_ref[...])


def _stage_a1(features, wiT, bi, wjT, bj):
    grid = (N // BN,)
    return pl.pallas_call(
        _a1_body,
        grid=grid,
        in_specs=[
            pl.BlockSpec((BN, D), lambda i: (i, 0)),
            pl.BlockSpec((D, D), lambda i: (0, 0)),
            pl.BlockSpec((1, D), lambda i: (0, 0)),
            pl.BlockSpec((D, D), lambda i: (0, 0)),
            pl.BlockSpec((1, D), lambda i: (0, 0)),
        ],
        out_specs=[
            pl.BlockSpec((BN, D), lambda i: (i, 0)),
            pl.BlockSpec((BN, D), lambda i: (i, 0)),
        ],
        out_shape=[
            jax.ShapeDtypeStruct((N, D), jnp.float32),
            jax.ShapeDtypeStruct((N, D), jnp.float32),
        ],
    )(features, wiT, bi, wjT, bj)


def _a2_body(dT_ref, wglo_ref, wghi_ref, g_ref):
    dT = dT_ref[...]
    glo = jax.lax.dot_general(dT, wglo_ref[...], (((0,), (1,)), ((), ())),
                              preferred_element_type=jnp.float32)
    ghi = jax.lax.dot_general(dT, wghi_ref[...], (((0,), (1,)), ((), ())),
                              preferred_element_type=jnp.float32)
    g_ref[...] = _pack_bf16_pair(glo, ghi)


def _stage_a2(descriptorsT, Wg):
    grid = (E // BE,)
    h = D // 2
    return pl.pallas_call(
        _a2_body,
        grid=grid,
        in_specs=[
            pl.BlockSpec((R, BE), lambda i: (0, i)),
            pl.BlockSpec((h, R), lambda i: (0, 0)),
            pl.BlockSpec((h, R), lambda i: (1, 0)),
        ],
        out_specs=pl.BlockSpec((BE, h), lambda i: (i, 0)),
        out_shape=jax.ShapeDtypeStruct((E, h), jnp.int32),
    )(descriptorsT, Wg, Wg)


def _sc_edge_body(g_hbm, xjd_hbm, idxj_hbm, idxi_hbm, out_hbm,
                  g_v0, g_v1, rows_v0, rows_v1, prod,
                  idxj_v0, idxj_v1, idxi_v0, idxi_v1,
                  jt, it, acc,
                  sg0, sg1, sr0, sr1, sj0, sj1, si0, si1):
    c = lax.axis_index("c")
    s = lax.axis_index("s")

    gv = (g_v0, g_v1)
    rv = (rows_v0, rows_v1)
    jv = (idxj_v0, idxj_v1)
    iv = (idxi_v0, idxi_v1)
    sg = (sg0, sg1)
    sr = (sr0, sr1)
    sj = (sj0, sj1)
    si = (si0, si1)

    def _zrow(e, carry):
        for q in range(D // 16):
            prod[e, pl.ds(q * 16, 16)] = jnp.zeros((16,), jnp.float32)
        return carry
    lax.fori_loop(0, CHUNK, _zrow, 0)
    for k in range(RPT // CHUNK):
        pltpu.sync_copy(prod, acc.at[pl.ds(s * RPT + k * CHUNK, CHUNK)])
    nfull = (RPT // CHUNK) * CHUNK
    pltpu.sync_copy(prod.at[pl.ds(0, RPT - nfull)],
                    acc.at[pl.ds(s * RPT + nfull, RPT - nfull)])

    @pl.when(s == 0)
    def _zero_rem():
        pltpu.sync_copy(prod.at[pl.ds(0, REM)], acc.at[pl.ds(NS * RPT, REM)])

    plsc.subcore_barrier()

    wbase = (c * NS + s) * EPW

    def _ds(kk):
        kkw = jnp.minimum(kk, NCHUNK - 1)
        return pl.ds(wbase + kkw * CHUNK, CHUNK)

    himask = jax.lax.broadcast(jnp.int32(-65536), (16,))

    def _mul_edge(gref, rref, dst_e, e):
        for q in range(D // 32):
            ds16 = pl.ds(q * 16, 16)
            gw = gref[e, ds16]
            glo = jax.lax.bitcast_convert_type(
                jax.lax.shift_left(gw, 16), jnp.float32)
            ghi = jax.lax.bitcast_convert_type(gw & himask, jnp.float32)
            rlo = rref[e, pl.ds(q * 16, 16)]
            rhi = rref[e, pl.ds(64 + q * 16, 16)]
            prod[dst_e, pl.ds(q * 16, 16)] = glo * rlo
            prod[dst_e, pl.ds(64 + q * 16, 16)] = ghi * rhi

    def _mul_chunk(b):
        def _mul(e, cc):
            _mul_edge(gv[b], rv[b], e, e)
            return cc
        lax.fori_loop(0, CHUNK, _mul, 0)

    def _step(k, b):
        nb = 1 - b
        pltpu.make_async_copy(idxj_hbm.at[_ds(k + 1)], jv[nb], sj[nb]).wait()
        pltpu.async_copy(g_hbm.at[_ds(k + 1)], gv[nb], sg[nb])
        pltpu.async_copy(xjd_hbm.at[jv[nb]], rv[nb], sr[nb])
        pltpu.make_async_copy(g_hbm.at[_ds(k)], gv[b], sg[b]).wait()
        pltpu.make_async_copy(xjd_hbm.at[jv[b]], rv[b], sr[b]).wait()
        pltpu.async_copy(idxj_hbm.at[_ds(k + 2)], jv[b], sj[b])
        _mul_chunk(b)
        pltpu.make_async_copy(idxi_hbm.at[_ds(k)], iv[b], si[b]).wait()
        pltpu.sync_copy(prod, acc.at[iv[b]], add=True)
        pltpu.async_copy(idxi_hbm.at[_ds(k + 2)], iv[b], si[b])

    pltpu.async_copy(idxj_hbm.at[_ds(0)], jv[0], sj[0])
    pltpu.async_copy(idxi_hbm.at[_ds(0)], iv[0], si[0])
    pltpu.async_copy(idxj_hbm.at[_ds(1)], jv[1], sj[1])
    pltpu.async_copy(idxi_hbm.at[_ds(1)], iv[1], si[1])
    pltpu.make_async_copy(idxj_hbm.at[_ds(0)], jv[0], sj[0]).wait()
    pltpu.async_copy(g_hbm.at[_ds(0)], gv[0], sg[0])
    pltpu.async_copy(xjd_hbm.at[jv[0]], rv[0], sr[0])

    def _pair(t, carry):
        _step(2 * t, 0)
        _step(2 * t + 1, 1)
        return carry
    lax.fori_loop(0, NCHUNK // 2, _pair, 0)

    pltpu.make_async_copy(g_hbm.at[_ds(0)], gv[0], sg[0]).wait()
    pltpu.make_async_copy(xjd_hbm.at[jv[0]], rv[0], sr[0]).wait()
    pltpu.make_async_copy(idxj_hbm.at[_ds(0)], jv[1], sj[1]).wait()
    pltpu.make_async_copy(idxi_hbm.at[_ds(0)], iv[0], si[0]).wait()
    pltpu.make_async_copy(idxi_hbm.at[_ds(0)], iv[1], si[1]).wait()

    wtail = wbase + NCHUNK * CHUNK
    pltpu.sync_copy(idxj_hbm.at[pl.ds(wtail, TAIL)], jt)
    pltpu.sync_copy(idxi_hbm.at[pl.ds(wtail, TAIL)], it)
    pltpu.sync_copy(g_hbm.at[pl.ds(wtail, TAIL)], gv[0].at[pl.ds(0, TAIL)])
    pltpu.async_copy(xjd_hbm.at[jt], rv[0].at[pl.ds(0, TAIL)], sr0).wait()

    def _mul_tail(e, cc):
        _mul_edge(gv[0], rv[0], e, e)
        return cc
    lax.fori_loop(0, TAIL, _mul_tail, 0)
    pltpu.sync_copy(prod.at[pl.ds(0, TAIL)], acc.at[it], add=True)

    plsc.subcore_barrier()

    rbase = s * RPT
    pltpu.sync_copy(acc.at[pl.ds(rbase, RPT)],
                    out_hbm.at[c, pl.ds(rbase, RPT)])

    @pl.when(s == 0)
    def _write_rem():
        pltpu.sync_copy(acc.at[pl.ds(NS * RPT, REM)],
                        out_hbm.at[c, pl.ds(NS * RPT, REM)])


def _stage_b(g, xjd, idx_j, idx_i):
    mesh = plsc.VectorSubcoreMesh(core_axis_name="c", subcore_axis_name="s",
                                  num_cores=NC, num_subcores=NS)
    fn = pl.kernel(
        _sc_edge_body,
        out_type=jax.ShapeDtypeStruct((NC, N, D), jnp.float32),
        mesh=mesh,
        scratch_types=[
            pltpu.VMEM((CHUNK, D // 2), jnp.int32),
            pltpu.VMEM((CHUNK, D // 2), jnp.int32),
            pltpu.VMEM((CHUNK, D), jnp.float32),
            pltpu.VMEM((CHUNK, D), jnp.float32),
            pltpu.VMEM((CHUNK, D), jnp.float32),
            pltpu.VMEM((CHUNK,), jnp.int32),
            pltpu.VMEM((CHUNK,), jnp.int32),
            pltpu.VMEM((CHUNK,), jnp.int32),
            pltpu.VMEM((CHUNK,), jnp.int32),
            pltpu.VMEM((TAIL,), jnp.int32),
            pltpu.VMEM((TAIL,), jnp.int32),
            pltpu.VMEM_SHARED((N, D), jnp.float32),
            pltpu.SemaphoreType.DMA,
            pltpu.SemaphoreType.DMA,
            pltpu.SemaphoreType.DMA,
            pltpu.SemaphoreType.DMA,
            pltpu.SemaphoreType.DMA,
            pltpu.SemaphoreType.DMA,
            pltpu.SemaphoreType.DMA,
            pltpu.SemaphoreType.DMA,
        ],
    )
    return fn(g, xjd, idx_j, idx_i)


def _c_body(xi_ref, p_ref, f_ref,
            riW1_ref, rib1_ref, riW2_ref, rib2_ref,
            wd_ref, bd_ref, u_ref,
            raW1_ref, rab1_ref, raW2_ref, rab2_ref,
            out_ref):
    m = xi_ref[...] + p_ref[0] + p_ref[1]
    for i in range(riW1_ref.shape[0]):
        y = _silu(m)
        t = _silu(_dot_t(y, riW1_ref[i]) + rib1_ref[i])
        m = m + _dot_t(t, riW2_ref[i]) + rib2_ref[i]
    m = _silu(m)
    x = u_ref[...] * f_ref[...] + _dot_t(m, wd_ref[...]) + bd_ref[...]
    for i in range(raW1_ref.shape[0]):
        y = _silu(x)
        t = _silu(_dot_t(y, raW1_ref[i]) + rab1_ref[i])
        x = x + _dot_t(t, raW2_ref[i]) + rab2_ref[i]
    out_ref[...] = x


def _stage_c(xi, p, features, riW1, rib1, riW2, rib2, wd, bd, u,
             raW1, rab1, raW2, rab2):
    grid = (N // BN,)
    nri = riW1.shape[0]
    nra = raW1.shape[0]
    return pl.pallas_call(
        _c_body,
        grid=grid,
        in_specs=[
            pl.BlockSpec((BN, D), lambda i: (i, 0)),
            pl.BlockSpec((NC, BN, D), lambda i: (0, i, 0)),
            pl.BlockSpec((BN, D), lambda i: (i, 0)),
            pl.BlockSpec((nri, D, D), lambda i: (0, 0, 0)),
            pl.BlockSpec((nri, 1, D), lambda i: (0, 0, 0)),
            pl.BlockSpec((nri, D, D), lambda i: (0, 0, 0)),
            pl.BlockSpec((nri, 1, D), lambda i: (0, 0, 0)),
            pl.BlockSpec((D, D), lambda i: (0, 0)),
            pl.BlockSpec((1, D), lambda i: (0, 0)),
            pl.BlockSpec((1, D), lambda i: (0, 0)),
            pl.BlockSpec((nra, D, D), lambda i: (0, 0, 0)),
            pl.BlockSpec((nra, 1, D), lambda i: (0, 0, 0)),
            pl.BlockSpec((nra, D, D), lambda i: (0, 0, 0)),
            pl.BlockSpec((nra, 1, D), lambda i: (0, 0, 0)),
        ],
        out_specs=pl.BlockSpec((BN, D), lambda i: (i, 0)),
        out_shape=jax.ShapeDtypeStruct((N, D), jnp.float32),
    )(xi, p, features, riW1, rib1, riW2, rib2, wd, bd, u,
      raW1, rab1, raW2, rab2)


def kernel(features, descriptors, idx_i, idx_j, Wg, Wi, bi, Wj, bj,
           ri_W1, ri_b1, ri_W2, ri_b2, Wd, bd, u, ra_W1, ra_b1, ra_W2, ra_b2):
    bi2 = bi.reshape(1, D)
    bd2 = bd.reshape(1, D)
    u2 = u.reshape(1, D)
    rib1 = ri_b1.reshape(-1, 1, D)
    rib2 = ri_b2.reshape(-1, 1, D)
    rab1 = ra_b1.reshape(-1, 1, D)
    rab2 = ra_b2.reshape(-1, 1, D)

    xi, xjd = _stage_a1(features, Wi, bi2, Wj, bj.reshape(1, D))
    g = _stage_a2(descriptors.T, Wg)
    p = _stage_b(g, xjd, idx_j, idx_i)
    return _stage_c(xi, p, features, ri_W1, rib1, ri_W2, rib2, Wd, bd2, u2,
                    ra_W1, rab1, ra_W2, rab2)


# speedup vs baseline: 6.1982x; 1.0092x over previous
import functools

import numpy as np

import jax
import jax.numpy as jnp
from jax import lax
from jax.experimental import pallas as pl
from jax.experimental.pallas import tpu as pltpu
from jax.experimental.pallas import tpu_sc as plsc

N = 10000
E = 320000
D = 128
R = 64

NC = 2
NS = 16
NW = NC * NS
EPW = E // NW
CHUNK = 72
NCHUNK = 138
TAIL = EPW - NCHUNK * CHUNK
RPT = 624
REM = N - NS * RPT

BN = 2000
BE = 12800


def _silu(x):
    return x * jax.nn.sigmoid(x)


def _dot_t(x, w):
    return jax.lax.dot_general(x, w, (((1,), (1,)), ((), ())),
                               preferred_element_type=jnp.float32)


def _pack_bf16_pair(lo_f32, hi_f32):
    hi_bits = jax.lax.bitcast_convert_type(
        hi_f32.astype(jnp.bfloat16).astype(jnp.float32), jnp.int32)
    lo_bits = jax.lax.shift_right_logical(
        jax.lax.bitcast_convert_type(
            lo_f32.astype(jnp.bfloat16).astype(jnp.float32), jnp.int32),
        jnp.int32(16))
    return hi_bits | lo_bits


def _a1_body(f_ref, wi_ref, bi_ref, wj_ref, bj_ref, xi_ref, xjd_ref):
    xa = _silu(f_ref[...])
    xi_ref[...] = _silu(_dot_t(xa, wi_ref[...]) + bi_ref[...])
    xjd_ref[...] = _silu(_dot_t(xa, wj_ref[...]) + bj_ref[...])


def _stage_a1(features, wiT, bi, wjT, bj):
    grid = (N // BN,)
    return pl.pallas_call(
        _a1_body,
        grid=grid,
        in_specs=[
            pl.BlockSpec((BN, D), lambda i: (i, 0)),
            pl.BlockSpec((D, D), lambda i: (0, 0)),
            pl.BlockSpec((1, D), lambda i: (0, 0)),
            pl.BlockSpec((D, D), lambda i: (0, 0)),
            pl.BlockSpec((1, D), lambda i: (0, 0)),
        ],
        out_specs=[
            pl.BlockSpec((BN, D), lambda i: (i, 0)),
            pl.BlockSpec((BN, D), lambda i: (i, 0)),
        ],
        out_shape=[
            jax.ShapeDtypeStruct((N, D), jnp.float32),
            jax.ShapeDtypeStruct((N, D), jnp.float32),
        ],
    )(features, wiT, bi, wjT, bj)


def _a2_body(dT_ref, wglo_ref, wghi_ref, g_ref):
    dT = dT_ref[...]
    glo = jax.lax.dot_general(dT, wglo_ref[...], (((0,), (1,)), ((), ())),
                              preferred_element_type=jnp.float32)
    ghi = jax.lax.dot_general(dT, wghi_ref[...], (((0,), (1,)), ((), ())),
                              preferred_element_type=jnp.float32)
    g_ref[...] = _pack_bf16_pair(glo, ghi)


def _stage_a2(descriptorsT, Wg):
    grid = (E // BE,)
    h = D // 2
    return pl.pallas_call(
        _a2_body,
        grid=grid,
        in_specs=[
            pl.BlockSpec((R, BE), lambda i: (0, i)),
            pl.BlockSpec((h, R), lambda i: (0, 0)),
            pl.BlockSpec((h, R), lambda i: (1, 0)),
        ],
        out_specs=pl.BlockSpec((BE, h), lambda i: (i, 0)),
        out_shape=jax.ShapeDtypeStruct((E, h), jnp.int32),
    )(descriptorsT, Wg, Wg)


def _sc_edge_body(g_hbm, xjd_hbm, idxj_hbm, idxi_hbm, out_hbm,
                  g_v0, g_v1, rows_v0, rows_v1, prod,
                  idxj_v0, idxj_v1, idxi_v0, idxi_v1,
                  jt, it, acc,
                  sg0, sg1, sr0, sr1, sj0, sj1, si0, si1, ss):
    c = lax.axis_index("c")
    s = lax.axis_index("s")

    gv = (g_v0, g_v1)
    rv = (rows_v0, rows_v1)
    jv = (idxj_v0, idxj_v1)
    iv = (idxi_v0, idxi_v1)
    sg = (sg0, sg1)
    sr = (sr0, sr1)
    sj = (sj0, sj1)
    si = (si0, si1)

    def _zrow(e, carry):
        for q in range(D // 16):
            prod[e, pl.ds(q * 16, 16)] = jnp.zeros((16,), jnp.float32)
        return carry
    lax.fori_loop(0, CHUNK, _zrow, 0)
    for k in range(RPT // CHUNK):
        pltpu.sync_copy(prod, acc.at[pl.ds(s * RPT + k * CHUNK, CHUNK)])
    nfull = (RPT // CHUNK) * CHUNK
    pltpu.sync_copy(prod.at[pl.ds(0, RPT - nfull)],
                    acc.at[pl.ds(s * RPT + nfull, RPT - nfull)])

    @pl.when(s == 0)
    def _zero_rem():
        pltpu.sync_copy(prod.at[pl.ds(0, REM)], acc.at[pl.ds(NS * RPT, REM)])

    plsc.subcore_barrier()

    wbase = (c * NS + s) * EPW

    def _ds(kk):
        kkw = jnp.minimum(kk, NCHUNK - 1)
        return pl.ds(wbase + kkw * CHUNK, CHUNK)

    himask = jax.lax.broadcast(jnp.int32(-65536), (16,))

    def _mul_edge(gref, rref, dst_e, e):
        for q in range(D // 32):
            ds16 = pl.ds(q * 16, 16)
            gw = gref[e, ds16]
            glo = jax.lax.bitcast_convert_type(
                jax.lax.shift_left(gw, 16), jnp.float32)
            ghi = jax.lax.bitcast_convert_type(gw & himask, jnp.float32)
            rlo = rref[e, pl.ds(q * 16, 16)]
            rhi = rref[e, pl.ds(64 + q * 16, 16)]
            prod[dst_e, pl.ds(q * 16, 16)] = glo * rlo
            prod[dst_e, pl.ds(64 + q * 16, 16)] = ghi * rhi

    def _mul_chunk(b):
        def _mul(e, cc):
            _mul_edge(gv[b], rv[b], e, e)
            return cc
        lax.fori_loop(0, CHUNK, _mul, 0)

    def _step(k, b):
        nb = 1 - b
        pltpu.make_async_copy(idxj_hbm.at[_ds(k + 1)], jv[nb], sj[nb]).wait()
        pltpu.async_copy(g_hbm.at[_ds(k + 1)], gv[nb], sg[nb])
        pltpu.async_copy(xjd_hbm.at[jv[nb]], rv[nb], sr[nb])
        pltpu.make_async_copy(g_hbm.at[_ds(k)], gv[b], sg[b]).wait()
        pltpu.make_async_copy(xjd_hbm.at[jv[b]], rv[b], sr[b]).wait()
        pltpu.async_copy(idxj_hbm.at[_ds(k + 2)], jv[b], sj[b])
        pltpu.make_async_copy(prod, acc.at[iv[nb]], ss).wait()
        _mul_chunk(b)
        pltpu.make_async_copy(idxi_hbm.at[_ds(k)], iv[b], si[b]).wait()
        pltpu.async_copy(prod, acc.at[iv[b]], ss, add=True)
        pltpu.async_copy(idxi_hbm.at[_ds(k + 2)], iv[b], si[b])

    pltpu.async_copy(idxj_hbm.at[_ds(0)], jv[0], sj[0])
    pltpu.async_copy(idxi_hbm.at[_ds(0)], iv[0], si[0])
    pltpu.sync_copy(idxi_hbm.at[_ds(1)], iv[1])
    pltpu.async_copy(prod, acc.at[iv[1]], ss, add=True)
    pltpu.async_copy(idxi_hbm.at[_ds(1)], iv[1], si[1])
    pltpu.async_copy(idxj_hbm.at[_ds(1)], jv[1], sj[1])
    pltpu.make_async_copy(idxj_hbm.at[_ds(0)], jv[0], sj[0]).wait()
    pltpu.async_copy(g_hbm.at[_ds(0)], gv[0], sg[0])
    pltpu.async_copy(xjd_hbm.at[jv[0]], rv[0], sr[0])

    def _pair(t, carry):
        _step(2 * t, 0)
        _step(2 * t + 1, 1)
        return carry
    lax.fori_loop(0, NCHUNK // 2, _pair, 0)

    pltpu.make_async_copy(g_hbm.at[_ds(0)], gv[0], sg[0]).wait()
    pltpu.make_async_copy(xjd_hbm.at[jv[0]], rv[0], sr[0]).wait()
    pltpu.make_async_copy(idxj_hbm.at[_ds(0)], jv[1], sj[1]).wait()
    pltpu.make_async_copy(idxi_hbm.at[_ds(0)], iv[0], si[0]).wait()
    pltpu.make_async_copy(idxi_hbm.at[_ds(0)], iv[1], si[1]).wait()
    pltpu.make_async_copy(prod, acc.at[iv[1]], ss).wait()

    wtail = wbase + NCHUNK * CHUNK
    pltpu.sync_copy(idxj_hbm.at[pl.ds(wtail, TAIL)], jt)
    pltpu.sync_copy(idxi_hbm.at[pl.ds(wtail, TAIL)], it)
    pltpu.sync_copy(g_hbm.at[pl.ds(wtail, TAIL)], gv[0].at[pl.ds(0, TAIL)])
    pltpu.async_copy(xjd_hbm.at[jt], rv[0].at[pl.ds(0, TAIL)], sr0).wait()

    def _mul_tail(e, cc):
        _mul_edge(gv[0], rv[0], e, e)
        return cc
    lax.fori_loop(0, TAIL, _mul_tail, 0)
    pltpu.sync_copy(prod.at[pl.ds(0, TAIL)], acc.at[it], add=True)

    plsc.subcore_barrier()

    rbase = s * RPT
    pltpu.sync_copy(acc.at[pl.ds(rbase, RPT)],
                    out_hbm.at[c, pl.ds(rbase, RPT)])

    @pl.when(s == 0)
    def _write_rem():
        pltpu.sync_copy(acc.at[pl.ds(NS * RPT, REM)],
                        out_hbm.at[c, pl.ds(NS * RPT, REM)])


def _stage_b(g, xjd, idx_j, idx_i):
    mesh = plsc.VectorSubcoreMesh(core_axis_name="c", subcore_axis_name="s",
                                  num_cores=NC, num_subcores=NS)
    fn = pl.kernel(
        _sc_edge_body,
        out_type=jax.ShapeDtypeStruct((NC, N, D), jnp.float32),
        mesh=mesh,
        scratch_types=[
            pltpu.VMEM((CHUNK, D // 2), jnp.int32),
            pltpu.VMEM((CHUNK, D // 2), jnp.int32),
            pltpu.VMEM((CHUNK, D), jnp.float32),
            pltpu.VMEM((CHUNK, D), jnp.float32),
            pltpu.VMEM((CHUNK, D), jnp.float32),
            pltpu.VMEM((CHUNK,), jnp.int32),
            pltpu.VMEM((CHUNK,), jnp.int32),
            pltpu.VMEM((CHUNK,), jnp.int32),
            pltpu.VMEM((CHUNK,), jnp.int32),
            pltpu.VMEM((TAIL,), jnp.int32),
            pltpu.VMEM((TAIL,), jnp.int32),
            pltpu.VMEM_SHARED((N, D), jnp.float32),
            pltpu.SemaphoreType.DMA,
            pltpu.SemaphoreType.DMA,
            pltpu.SemaphoreType.DMA,
            pltpu.SemaphoreType.DMA,
            pltpu.SemaphoreType.DMA,
            pltpu.SemaphoreType.DMA,
            pltpu.SemaphoreType.DMA,
            pltpu.SemaphoreType.DMA,
            pltpu.SemaphoreType.DMA,
        ],
    )
    return fn(g, xjd, idx_j, idx_i)


def _c_body(xi_ref, p_ref, f_ref,
            riW1_ref, rib1_ref, riW2_ref, rib2_ref,
            wd_ref, bd_ref, u_ref,
            raW1_ref, rab1_ref, raW2_ref, rab2_ref,
            out_ref):
    m = xi_ref[...] + p_ref[0] + p_ref[1]
    for i in range(riW1_ref.shape[0]):
        y = _silu(m)
        t = _silu(_dot_t(y, riW1_ref[i]) + rib1_ref[i])
        m = m + _dot_t(t, riW2_ref[i]) + rib2_ref[i]
    m = _silu(m)
    x = u_ref[...] * f_ref[...] + _dot_t(m, wd_ref[...]) + bd_ref[...]
    for i in range(raW1_ref.shape[0]):
        y = _silu(x)
        t = _silu(_dot_t(y, raW1_ref[i]) + rab1_ref[i])
        x = x + _dot_t(t, raW2_ref[i]) + rab2_ref[i]
    out_ref[...] = x


def _stage_c(xi, p, features, riW1, rib1, riW2, rib2, wd, bd, u,
             raW1, rab1, raW2, rab2):
    grid = (N // BN,)
    nri = riW1.shape[0]
    nra = raW1.shape[0]
    return pl.pallas_call(
        _c_body,
        grid=grid,
        in_specs=[
            pl.BlockSpec((BN, D), lambda i: (i, 0)),
            pl.BlockSpec((NC, BN, D), lambda i: (0, i, 0)),
            pl.BlockSpec((BN, D), lambda i: (i, 0)),
            pl.BlockSpec((nri, D, D), lambda i: (0, 0, 0)),
            pl.BlockSpec((nri, 1, D), lambda i: (0, 0, 0)),
            pl.BlockSpec((nri, D, D), lambda i: (0, 0, 0)),
            pl.BlockSpec((nri, 1, D), lambda i: (0, 0, 0)),
            pl.BlockSpec((D, D), lambda i: (0, 0)),
            pl.BlockSpec((1, D), lambda i: (0, 0)),
            pl.BlockSpec((1, D), lambda i: (0, 0)),
            pl.BlockSpec((nra, D, D), lambda i: (0, 0, 0)),
            pl.BlockSpec((nra, 1, D), lambda i: (0, 0, 0)),
            pl.BlockSpec((nra, D, D), lambda i: (0, 0, 0)),
            pl.BlockSpec((nra, 1, D), lambda i: (0, 0, 0)),
        ],
        out_specs=pl.BlockSpec((BN, D), lambda i: (i, 0)),
        out_shape=jax.ShapeDtypeStruct((N, D), jnp.float32),
    )(xi, p, features, riW1, rib1, riW2, rib2, wd, bd, u,
      raW1, rab1, raW2, rab2)


def kernel(features, descriptors, idx_i, idx_j, Wg, Wi, bi, Wj, bj,
           ri_W1, ri_b1, ri_W2, ri_b2, Wd, bd, u, ra_W1, ra_b1, ra_W2, ra_b2):
    bi2 = bi.reshape(1, D)
    bd2 = bd.reshape(1, D)
    u2 = u.reshape(1, D)
    rib1 = ri_b1.reshape(-1, 1, D)
    rib2 = ri_b2.reshape(-1, 1, D)
    rab1 = ra_b1.reshape(-1, 1, D)
    rab2 = ra_b2.reshape(-1, 1, D)

    xi, xjd = _stage_a1(features, Wi, bi2, Wj, bj.reshape(1, D))
    g = _stage_a2(descriptors.T, Wg)
    p = _stage_b(g, xjd, idx_j, idx_i)
    return _stage_c(xi, p, features, ri_W1, rib1, ri_W2, rib2, Wd, bd2, u2,
                    ra_W1, rab1, ra_W2, rab2)
